# baseline (speedup 1.0000x reference)
import functools

import jax
import jax.numpy as jnp
from jax import lax
from jax.experimental import pallas as pl
from jax.experimental.pallas import tpu as pltpu
from jax.experimental.pallas import tpu_sc as plsc

B = 1024
CTX = 20
EMB = 64
VOC = 100000

NC = 2
NS = 16
NW = NC * NS
BPW = B // NW
ROWS = BPW * CTX
IDX_CHUNK = 128
NCHUNK = ROWS // IDX_CHUNK

TN = 4096
NJ = pl.cdiv(VOC, TN)


CPW = EMB // NW


def _sc_pool_t_body(emb_u, idx_u, out_hbm, idx_v, row_v, out_v, sem):
    wid = lax.axis_index("s") * NC + lax.axis_index("c")
    pltpu.sync_copy(idx_u, idx_v)
    inv = jnp.float32(1.0 / CTX)

    for comp in range(CPW):
        e = wid * CPW + comp
        pltpu.sync_copy(emb_u.at[e], row_v)

        def body(bb, carry):
            acc = jnp.zeros((16,), jnp.float32)
            for c in range(CTX):
                ivec = idx_v[c, pl.ds(bb * 16, 16)]
                acc = acc + plsc.load_gather(row_v, [ivec])
            out_v[pl.ds(bb * 16, 16)] = acc * inv
            return carry

        lax.fori_loop(0, B // 16, body, 0)
        pltpu.sync_copy(out_v, out_hbm.at[e])


def _sc_pool_t(emb_u, idx_u):
    kern = pl.kernel(
        _sc_pool_t_body,
        out_type=jax.ShapeDtypeStruct((EMB, B), jnp.float32),
        mesh=plsc.VectorSubcoreMesh(core_axis_name="c", subcore_axis_name="s"),
        scratch_types=[
            pltpu.VMEM((CTX, B), jnp.int32),
            pltpu.VMEM((VOC,), jnp.float32),
            pltpu.VMEM((B,), jnp.float32),
            pltpu.SemaphoreType.DMA,
        ],
        compiler_params=pltpu.CompilerParams(
            use_tc_tiling_on_sc=False,
            needs_layout_passes=False,
        ),
    )
    return kern(emb_u, idx_u)


def _mm_body(w_ref, p_ref, o_ref):
    o_ref[...] = lax.dot_general(
        w_ref[...],
        p_ref[...],
        dimension_numbers=(((0,), (0,)), ((), ())),
        preferred_element_type=jnp.float32,
    )


def _tc_matmul_t(ffw_t, pooled_t):
    return pl.pallas_call(
        _mm_body,
        grid=(NJ,),
        in_specs=[
            pl.BlockSpec((EMB, TN), lambda j: (0, j)),
            pl.BlockSpec((EMB, B), lambda j: (0, 0)),
        ],
        out_specs=pl.BlockSpec((TN, B), lambda j: (j, 0)),
        out_shape=jax.ShapeDtypeStruct((VOC, B), jnp.float32),
        compiler_params=pltpu.CompilerParams(
            dimension_semantics=("parallel",),
            vmem_limit_bytes=100 * 1024 * 1024,
        ),
    )(ffw_t, pooled_t)


def kernel(inpt, emb_table, ffw_weight):
    idx_u = inpt.astype(jnp.int32).T
    pooled_t = _sc_pool_t(emb_table.T, idx_u)
    out_t = _tc_matmul_t(ffw_weight.T, pooled_t)
    return out_t.T


# speedup vs baseline: 1.0109x; 1.0109x over previous
import jax
import jax.numpy as jnp
from jax import lax
from jax.experimental import pallas as pl
from jax.experimental.pallas import tpu as pltpu
from jax.experimental.pallas import tpu_sc as plsc

B = 1024
CTX = 20
EMB = 64
VOC = 100000

NC = 2
NS = 16
NW = NC * NS
TN = 4096
NJ = pl.cdiv(VOC, TN)


CPW = EMB // NW


def _sc_pool_t_body(emb_u, idx_u, out_hbm, idx_v, row_v, out_v, sem):
    wid = lax.axis_index("s") * NC + lax.axis_index("c")
    pltpu.sync_copy(idx_u, idx_v)
    inv = jnp.float32(1.0 / CTX)

    for comp in range(CPW):
        e = wid * CPW + comp
        pltpu.sync_copy(emb_u.at[e], row_v)

        def body(bb, carry):
            acc = jnp.zeros((16,), jnp.float32)
            for c in range(CTX):
                ivec = idx_v[c, pl.ds(bb * 16, 16)]
                acc = acc + plsc.load_gather(row_v, [ivec])
            out_v[pl.ds(bb * 16, 16)] = acc * inv
            return carry

        lax.fori_loop(0, B // 16, body, 0)
        pltpu.sync_copy(out_v, out_hbm.at[e])


def _sc_pool_t(emb_u, idx_u):
    kern = pl.kernel(
        _sc_pool_t_body,
        out_type=jax.ShapeDtypeStruct((EMB, B), jnp.float32),
        mesh=plsc.VectorSubcoreMesh(core_axis_name="c", subcore_axis_name="s"),
        scratch_types=[
            pltpu.VMEM((CTX, B), jnp.int32),
            pltpu.VMEM((VOC,), jnp.float32),
            pltpu.VMEM((B,), jnp.float32),
            pltpu.SemaphoreType.DMA,
        ],
        compiler_params=pltpu.CompilerParams(
            use_tc_tiling_on_sc=False,
            needs_layout_passes=False,
        ),
    )
    return kern(emb_u, idx_u)


def _mm_body(w_ref, p_ref, o_ref):
    o_ref[...] = lax.dot_general(
        w_ref[...],
        p_ref[...],
        dimension_numbers=(((0,), (0,)), ((), ())),
        preferred_element_type=jnp.float32,
    )


def _tc_matmul_t(ffw_t, pooled_t):
    return pl.pallas_call(
        _mm_body,
        grid=(NJ,),
        in_specs=[
            pl.BlockSpec((EMB, TN), lambda j: (0, j)),
            pl.BlockSpec((EMB, B), lambda j: (0, 0)),
        ],
        out_specs=pl.BlockSpec((TN, B), lambda j: (j, 0)),
        out_shape=jax.ShapeDtypeStruct((VOC, B), jnp.float32),
        compiler_params=pltpu.CompilerParams(
            dimension_semantics=("arbitrary",),
            vmem_limit_bytes=100 * 1024 * 1024,
        ),
    )(ffw_t, pooled_t)


def kernel(inpt, emb_table, ffw_weight):
    idx_u = inpt.astype(jnp.int32).T
    pooled_t = _sc_pool_t(emb_table.T, idx_u)
    out_t = _tc_matmul_t(ffw_weight.T, pooled_t)
    return out_t.T
